# baseline (speedup 1.0000x reference)
import functools

import jax
import jax.numpy as jnp
import numpy as np
from jax import lax
from jax.experimental import pallas as pl
from jax.experimental.pallas import tpu as pltpu
from jax.experimental.pallas import tpu_sc as plsc

_MAX_POS = 2048
_EPS = 1e-12
_NC, _NS, _LANES = 2, 16, 16
_NW = _NC * _NS


def _pe_table(dim):
    position = np.arange(_MAX_POS, dtype=np.float32)[:, None]
    div_term = np.exp(
        np.arange(0, dim, 2, dtype=np.float32) * -(np.log(10000.0) / dim))
    pe = np.zeros((_MAX_POS, dim), dtype=np.float32)
    pe[:, 0::2] = np.sin(position * div_term)
    pe[:, 1::2] = np.cos(position * div_term)
    return jnp.asarray(pe)


@functools.lru_cache(maxsize=None)
def _make_gather_sum(T, D):
    CPW = T // _NW
    C = 8
    NCH = CPW // C
    NP = NCH // 2
    NV = D // _LANES
    mesh = plsc.VectorSubcoreMesh(
        core_axis_name="c", subcore_axis_name="s",
        num_cores=_NC, num_subcores=_NS)

    buf = pltpu.VMEM((C, D), jnp.float32)
    idx = pltpu.VMEM((CPW,), jnp.int32)

    @functools.partial(
        pl.kernel,
        out_type=jax.ShapeDtypeStruct((T, D), jnp.float32),
        mesh=mesh,
        scratch_types=(
            [idx] * 4 + [buf] * 10 + [pltpu.SemaphoreType.DMA] * 4),
    )
    def gather_sum(ids_h, para_h, sent_h, tok_h, wemb_h, pe_h, out_h,
                   ids_v, para_v, sent_v, tok_v,
                   bw0, bp0, bs0, bt0, ob0, bw1, bp1, bs1, bt1, ob1,
                   sg0, sg1, so0, so1):
        wid = lax.axis_index("s") * _NC + lax.axis_index("c")
        base = wid * CPW
        pltpu.sync_copy(ids_h.at[pl.ds(base, CPW)], ids_v)
        pltpu.sync_copy(para_h.at[pl.ds(base, CPW)], para_v)
        pltpu.sync_copy(sent_h.at[pl.ds(base, CPW)], sent_v)
        pltpu.sync_copy(tok_h.at[pl.ds(base, CPW)], tok_v)

        slots = [
            dict(bw=bw0, bp=bp0, bs=bs0, bt=bt0, ob=ob0, sg=sg0, so=so0),
            dict(bw=bw1, bp=bp1, bs=bs1, bt=bt1, ob=ob1, sg=sg1, so=so1),
        ]

        def g_copies(i, sl, make):
            f = pltpu.make_async_copy if make else pltpu.async_copy
            off = i * C
            return [
                f(wemb_h.at[ids_v.at[pl.ds(off, C)]], sl["bw"], sl["sg"]),
                f(pe_h.at[para_v.at[pl.ds(off, C)]], sl["bp"], sl["sg"]),
                f(pe_h.at[sent_v.at[pl.ds(off, C)]], sl["bs"], sl["sg"]),
                f(pe_h.at[tok_v.at[pl.ds(off, C)]], sl["bt"], sl["sg"]),
            ]

        def fire_g(i, sl):
            g_copies(i, sl, False)

        def wait_g(i, sl):
            for d in g_copies(i, sl, True):
                d.wait()

        def fire_o(i, sl):
            pltpu.async_copy(
                sl["ob"], out_h.at[pl.ds(base + i * C, C)], sl["so"])

        def wait_o(i, sl):
            pltpu.make_async_copy(
                sl["ob"], out_h.at[pl.ds(base + i * C, C)], sl["so"]).wait()

        def compute(sl):
            bw, bp, bs, bt, ob = (
                sl["bw"], sl["bp"], sl["bs"], sl["bt"], sl["ob"])

            def row(t, c):
                for j in range(NV):
                    s_ = pl.ds(j * _LANES, _LANES)
                    ob[t, s_] = bw[t, s_] + bp[t, s_] + bs[t, s_] + bt[t, s_]
                return c
            lax.fori_loop(0, C, row, 0)

        def pair(p, first, last):
            for s in range(2):
                i = 2 * p + s
                sl = slots[s]
                wait_g(i, sl)
                if not first:
                    wait_o(i - 2, sl)
                compute(sl)
                fire_o(i, sl)
                if not last:
                    fire_g(i + 2, sl)

        fire_g(0, slots[0])
        fire_g(1, slots[1])
        pair(0, True, False)

        def mid(p, c):
            pair(p, False, False)
            return c
        lax.fori_loop(1, NP - 1, mid, 0)

        pair(NP - 1, False, True)
        wait_o(NCH - 2, slots[0])
        wait_o(NCH - 1, slots[1])

    return gather_sum


def _ln_body(acc_ref, pe_ref, tt_ref, te_ref, w_ref, b_ref, out_ref):
    x = acc_ref[...] + pe_ref[...]
    t = tt_ref[...]
    x = x + te_ref[0:1, :] + t * (te_ref[1:2, :] - te_ref[0:1, :])
    mu = jnp.mean(x, axis=-1, keepdims=True)
    xc = x - mu
    var = jnp.mean(xc * xc, axis=-1, keepdims=True)
    out_ref[...] = xc * lax.rsqrt(var + _EPS) * w_ref[...] + b_ref[...]


@functools.lru_cache(maxsize=None)
def _make_ln(T, N, D):
    R = 256
    nb_pe = N // R
    return pl.pallas_call(
        _ln_body,
        grid=(T // R,),
        in_specs=[
            pl.BlockSpec((R, D), lambda i: (i, 0)),
            pl.BlockSpec((R, D), lambda i: (i % nb_pe, 0)),
            pl.BlockSpec((R, 1), lambda i: (i, 0)),
            pl.BlockSpec((2, D), lambda i: (0, 0)),
            pl.BlockSpec((1, D), lambda i: (0, 0)),
            pl.BlockSpec((1, D), lambda i: (0, 0)),
        ],
        out_specs=pl.BlockSpec((R, D), lambda i: (i, 0)),
        out_shape=jax.ShapeDtypeStruct((T, D), jnp.float32),
    )


def kernel(input_ids, tok_struct_vec, token_type_ids, word_emb, type_emb,
           ln_weight, ln_bias):
    B, N = input_ids.shape
    D = word_emb.shape[1]
    T = B * N
    pe = _pe_table(D)

    ids = input_ids.reshape(T).astype(jnp.int32)
    para = tok_struct_vec[..., 0].reshape(T).astype(jnp.int32)
    sent = tok_struct_vec[..., 1].reshape(T).astype(jnp.int32)
    tok = tok_struct_vec[..., 2].reshape(T).astype(jnp.int32)

    acc = _make_gather_sum(T, D)(ids, para, sent, tok, word_emb, pe)

    tt = token_type_ids.reshape(T, 1).astype(jnp.float32)
    out = _make_ln(T, N, D)(
        acc, pe[:N], tt, type_emb,
        ln_weight.reshape(1, D), ln_bias.reshape(1, D))
    return out.reshape(B, N, D)
